# baseline (speedup 1.0000x reference)
import functools

import jax
import jax.numpy as jnp
from jax.experimental import pallas as pl
from jax.experimental.pallas import tpu as pltpu

_NE = 1024
_D = 64
_R = 2048


def _vq_block(x_ref, wt_ref, wtm2_ref, irow_ref, icol_ref,
              q_ref, counts_ref, sse_ref):
    xb = x_ref[...]
    wtm2 = wtm2_ref[...]
    x2 = jnp.sum(xb * xb, axis=1, keepdims=True)
    w2 = 0.25 * jnp.sum(wtm2 * wtm2, axis=0, keepdims=True)
    mm2 = jax.lax.dot_general(xb, wtm2, (((1,), (0,)), ((), ())),
                              preferred_element_type=jnp.float32)
    d = (x2 + w2) + mm2
    lane = irow_ref[...]
    dmin = jnp.min(d, axis=1, keepdims=True)
    idx = jnp.min(jnp.where(d == dmin, lane, float(_NE)), axis=1,
                  keepdims=True)
    idx_t = jnp.transpose(idx, (1, 0))
    onehot_t = (icol_ref[...] == idx_t).astype(jnp.float32)
    qt = jax.lax.dot_general(wt_ref[...], onehot_t, (((1,), (0,)), ((), ())),
                             preferred_element_type=jnp.float32)
    q_ref[0] = qt[:, :_NE]
    q_ref[1] = qt[:, _NE:]
    cb = jnp.sum(onehot_t, axis=1, keepdims=True)
    sb = jnp.sum(dmin, axis=0, keepdims=True)

    @pl.when(pl.program_id(0) == 0)
    def _init():
        counts_ref[...] = cb
        sse_ref[...] = sb

    @pl.when(pl.program_id(0) != 0)
    def _acc():
        counts_ref[...] += cb
        sse_ref[...] += sb


@functools.partial(jax.jit, static_argnames=())
def kernel(x, W):
    B, C, H, Wd = x.shape
    n = B * H * Wd
    x_flat = jnp.transpose(x, (0, 2, 3, 1)).reshape(n, _D)
    wt = W.T
    wtm2 = -2.0 * wt
    irow = jnp.arange(_NE, dtype=jnp.float32).reshape(1, _NE)
    icol = jnp.arange(_NE, dtype=jnp.float32).reshape(_NE, 1)
    grid = n // _R
    rpb = _R // (H * Wd)
    qc, counts, sse = pl.pallas_call(
        _vq_block,
        grid=(grid,),
        in_specs=[
            pl.BlockSpec((_R, _D), lambda i: (i, 0)),
            pl.BlockSpec((_D, _NE), lambda i: (0, 0)),
            pl.BlockSpec((_D, _NE), lambda i: (0, 0)),
            pl.BlockSpec((1, _NE), lambda i: (0, 0)),
            pl.BlockSpec((_NE, 1), lambda i: (0, 0)),
        ],
        out_specs=[
            pl.BlockSpec((rpb, _D, H * Wd), lambda i: (i, 0, 0)),
            pl.BlockSpec((_NE, 1), lambda i: (0, 0)),
            pl.BlockSpec((1, 1), lambda i: (0, 0)),
        ],
        out_shape=[
            jax.ShapeDtypeStruct((B, C, H * Wd), jnp.float32),
            jax.ShapeDtypeStruct((_NE, 1), jnp.float32),
            jax.ShapeDtypeStruct((1, 1), jnp.float32),
        ],
        compiler_params=pltpu.CompilerParams(
            dimension_semantics=("arbitrary",),
        ),
    )(x_flat, wt, wtm2, irow, icol)
    quantized = qc.reshape(B, C, H, Wd)
    m = sse[0, 0] / (n * _D)
    loss = m + 0.25 * m
    avg_probs = counts[:, 0] / n
    perplexity = jnp.exp(-jnp.sum(avg_probs * jnp.log(avg_probs + 1e-10)))
    return (quantized, loss, perplexity)
